# baseline (speedup 1.0000x reference)
import functools

import jax
import jax.numpy as jnp
from jax import lax
from jax.experimental import pallas as pl
from jax.experimental.pallas import tpu as pltpu
from jax.experimental.pallas import tpu_sc as plsc

N = 10000
E = 320000
EE = E + N
ROWS = 2592
EE_PAD0 = ROWS * 128
ROWS_PER_TILE = ROWS // 32
SROWS = 2816
EE_PAD = SROWS * 128
OCH = 11
CH = 16
ACC_ROWS = 10112
ZROWS = ACC_ROWS // 16
BN = 1000

_mesh = plsc.VectorSubcoreMesh(core_axis_name="c", subcore_axis_name="s")


@functools.partial(
    pl.kernel,
    out_type=jax.ShapeDtypeStruct((2, 16, ZROWS, 16), jnp.float32),
    mesh=_mesh,
    scratch_types=[
        pltpu.VMEM((ROWS_PER_TILE, 128), jnp.int32),
        pltpu.VMEM((128, 16), jnp.float32),
        pltpu.VMEM_SHARED((ACC_ROWS, 16), jnp.float32),
    ],
)
def _deg_kernel(dst_hbm, ones_hbm, z16_hbm, out_hbm, dstv, onesv, acc):
    cid = lax.axis_index("c")
    sid = lax.axis_index("s")
    pltpu.sync_copy(z16_hbm, acc.at[pl.ds(ZROWS * sid, ZROWS)])
    plsc.subcore_barrier()
    pltpu.sync_copy(ones_hbm, onesv)
    wid = cid * 16 + sid
    pltpu.sync_copy(dst_hbm.at[wid], dstv)

    def step(j, _):
        pltpu.sync_copy(onesv, acc.at[dstv.at[j]], add=True)
        return ()

    lax.fori_loop(0, ROWS_PER_TILE, step, ())
    plsc.subcore_barrier()
    pltpu.sync_copy(acc.at[pl.ds(ZROWS * sid, ZROWS)], out_hbm.at[cid, sid])


@functools.partial(
    pl.kernel,
    out_type=jax.ShapeDtypeStruct((2, 16, ZROWS, 128), jnp.float32),
    mesh=_mesh,
    scratch_types=[
        pltpu.VMEM((CH, 128), jnp.int32),
        pltpu.VMEM((CH, 128), jnp.int32),
        pltpu.VMEM((128, 128), jnp.float32),
        pltpu.VMEM((128, 128), jnp.float32),
        pltpu.VMEM_SHARED((ACC_ROWS, 128), jnp.float32),
        pltpu.SemaphoreType.DMA,
        pltpu.SemaphoreType.DMA,
    ],
)
def _scatter_kernel(g_hbm, src_hbm, dst_hbm, z128_hbm, out_hbm,
                    srcv, dstv, rows0, rows1, acc, sem0, sem1):
    cid = lax.axis_index("c")
    sid = lax.axis_index("s")
    pltpu.sync_copy(z128_hbm, acc.at[pl.ds(ZROWS * sid, ZROWS)])
    plsc.subcore_barrier()

    def outer(o, _):
        pltpu.sync_copy(src_hbm.at[cid, sid, o], srcv)
        pltpu.sync_copy(dst_hbm.at[sid, o], dstv)

        def step(j, _):
            pltpu.sync_copy(g_hbm.at[srcv.at[j]], rows0)
            pltpu.sync_copy(rows0, acc.at[dstv.at[j]], add=True)
            return ()

        lax.fori_loop(0, CH, step, ())
        return ()

    lax.fori_loop(0, OCH, outer, ())
    plsc.subcore_barrier()
    pltpu.sync_copy(acc.at[pl.ds(ZROWS * sid, ZROWS)], out_hbm.at[cid, sid])


@functools.partial(
    pl.kernel,
    out_type=jax.ShapeDtypeStruct((4, 64, 128), jnp.float32),
    mesh=_mesh,
    scratch_types=[
        pltpu.VMEM((80, 125), jnp.int32),
        pltpu.VMEM((125, 128), jnp.float32),
        pltpu.VMEM_SHARED((64, 128), jnp.float32),
    ],
)
def _pool_kernel(h_hbm, batch_hbm, z128_hbm, out_hbm, bv, rowsv, acc):
    cid = lax.axis_index("c")
    sid = lax.axis_index("s")
    pltpu.sync_copy(batch_hbm, bv)
    for cc in range(2):
        c = cid + 2 * cc
        @pl.when(sid == 0)
        def _():
            pltpu.sync_copy(z128_hbm.at[pl.ds(0, 64)], acc)

        plsc.subcore_barrier()
        for r in range(5):
            blk = 5 * sid + r
            pltpu.sync_copy(h_hbm.at[c, blk], rowsv)
            pltpu.sync_copy(rowsv, acc.at[bv.at[blk]], add=True)
        plsc.subcore_barrier()

        @pl.when(sid == 0)
        def _():
            pltpu.sync_copy(acc, out_hbm.at[c])

        plsc.subcore_barrier()


def _dinv_block(p_ref):
    deg = p_ref[0, :, 0:1] + p_ref[1, :, 0:1]
    return lax.rsqrt(deg)


def _tc1_body(p_ref, x_ref, w_ref, o_ref):
    dinv = _dinv_block(p_ref)
    o_ref[0] = dinv * jnp.dot(x_ref[...], w_ref[...],
                              preferred_element_type=jnp.float32)


def _mm_layer1(p, x, W1):
    return pl.pallas_call(
        _tc1_body,
        grid=(N // BN, 2),
        in_specs=[
            pl.BlockSpec((2, BN, 16), lambda i, j: (0, i, 0)),
            pl.BlockSpec((BN, 128), lambda i, j: (i, 0)),
            pl.BlockSpec((128, 128), lambda i, j: (0, j)),
        ],
        out_specs=pl.BlockSpec((1, BN, 128), lambda i, j: (j, i, 0)),
        out_shape=jax.ShapeDtypeStruct((2, N, 128), jnp.float32),
    )(p, x, W1)


def _tc_mid_body(nk, p_ref, raw_ref, b_ref, w_ref, o_ref):
    k = pl.program_id(2)
    dinv = _dinv_block(p_ref)
    h = jnp.maximum(dinv * raw_ref[0] + b_ref[0], 0.0)
    part = jnp.dot(h, w_ref[...], preferred_element_type=jnp.float32)

    @pl.when(k == 0)
    def _():
        o_ref[0] = part

    @pl.when(k > 0)
    def _():
        o_ref[0] += part

    @pl.when(k == nk - 1)
    def _():
        o_ref[0] = dinv * o_ref[0]


def _mm_mid(p, rawp, b_r, W, nj, nk):
    return pl.pallas_call(
        functools.partial(_tc_mid_body, nk),
        grid=(N // BN, nj, nk),
        in_specs=[
            pl.BlockSpec((2, BN, 16), lambda i, j, k: (0, i, 0)),
            pl.BlockSpec((1, BN, 128), lambda i, j, k: (k, i, 0)),
            pl.BlockSpec((1, 1, 128), lambda i, j, k: (k, 0, 0)),
            pl.BlockSpec((128, 128), lambda i, j, k: (k, j)),
        ],
        out_specs=pl.BlockSpec((1, BN, 128), lambda i, j, k: (j, i, 0)),
        out_shape=jax.ShapeDtypeStruct((nj, N, 128), jnp.float32),
    )(p, rawp, b_r, W)


def _tc_fin_body(p_ref, raw_ref, b_ref, o_ref):
    dinv = _dinv_block(p_ref)
    o_ref[0] = jnp.maximum(dinv * raw_ref[0] + b_ref[0], 0.0)


def _mm_finish(p, rawp, b_r):
    return pl.pallas_call(
        _tc_fin_body,
        grid=(N // BN, 4),
        in_specs=[
            pl.BlockSpec((2, BN, 16), lambda i, j: (0, i, 0)),
            pl.BlockSpec((1, BN, 128), lambda i, j: (j, i, 0)),
            pl.BlockSpec((1, 1, 128), lambda i, j: (j, 0, 0)),
        ],
        out_specs=pl.BlockSpec((1, BN, 128), lambda i, j: (j, i, 0)),
        out_shape=jax.ShapeDtypeStruct((4, N, 128), jnp.float32),
    )(p, rawp, b_r)


def _fc_body(pf_ref, wfc_ref, bfc_ref, wout_ref, bout_ref, o_ref):
    h = jnp.maximum(
        jnp.dot(pf_ref[...], wfc_ref[...], preferred_element_type=jnp.float32)
        + bfc_ref[...], 0.0)
    o_ref[...] = (jnp.dot(h, wout_ref[...], preferred_element_type=jnp.float32)
                  + bout_ref[...])


def _fc_head(pf, Wfc, bfc, Wout, bout):
    return pl.pallas_call(
        _fc_body,
        out_shape=jax.ShapeDtypeStruct((64, 256), jnp.float32),
    )(pf, Wfc, bfc.reshape(1, 512), Wout, bout.reshape(1, 256))


def kernel(x, edge_index, batch, W1, b1, W2, b2, W3, b3, Wfc, bfc, Wout, bout):
    loop = jnp.arange(N, dtype=jnp.int32)
    pad = EE_PAD - EE
    src0 = jnp.concatenate([edge_index[0], loop, jnp.zeros((pad,), jnp.int32)])
    dst0 = jnp.concatenate([edge_index[1], loop, jnp.full((pad,), N, jnp.int32)])
    src2 = jnp.stack([src0, src0 + N]).reshape(2, 16, OCH, CH, 128)
    dst2 = dst0.reshape(16, OCH, CH, 128)
    dst_deg = dst0[:EE_PAD0].reshape(32, ROWS_PER_TILE, 128)
    batch2d = batch.reshape(80, 125)
    ones16 = jnp.ones((128, 16), jnp.float32)
    z16 = jnp.zeros((ZROWS, 16), jnp.float32)
    z128 = jnp.zeros((ZROWS, 128), jnp.float32)

    p = _deg_kernel(dst_deg, ones16, z16).reshape(2, ACC_ROWS, 16)

    def conv_scatter(g):
        pairs = [_scatter_kernel(g[2 * q:2 * q + 2].reshape(2 * N, 128),
                                 src2, dst2, z128).reshape(2, ACC_ROWS, 128)
                 for q in range(g.shape[0] // 2)]
        return jnp.concatenate(pairs) if len(pairs) > 1 else pairs[0]

    g1 = _mm_layer1(p, x, W1)
    raw1 = conv_scatter(g1)
    g2 = _mm_mid(p, raw1, b1.reshape(2, 1, 128), W2, 2, 2)
    raw2 = conv_scatter(g2)
    g3 = _mm_mid(p, raw2, b2.reshape(2, 1, 128), W3, 4, 2)
    raw3 = conv_scatter(g3)
    h3 = _mm_finish(p, raw3, b3.reshape(4, 1, 128))

    pooled = _pool_kernel(h3.reshape(4, 80, 125, 128), batch2d, z128)
    embedding = jnp.transpose(pooled, (1, 0, 2)).reshape(64, 512)
    output = _fc_head(embedding, Wfc, bfc, Wout, bout)
    return (embedding, output)


# speedup vs baseline: 5.0171x; 5.0171x over previous
import functools

import jax
import jax.numpy as jnp
from jax import lax
from jax.experimental import pallas as pl
from jax.experimental.pallas import tpu as pltpu
from jax.experimental.pallas import tpu_sc as plsc

N = 10000
E = 320000
EE = E + N
ROWS = 2592
EE_PAD0 = ROWS * 128
ROWS_PER_TILE = ROWS // 32
SROWS = 2592
EE_PAD = SROWS * 128
OCH = 9
CH = 18
ACC_ROWS = 10112
ZROWS = ACC_ROWS // 16
BN = 1000

_mesh = plsc.VectorSubcoreMesh(core_axis_name="c", subcore_axis_name="s")


@functools.partial(
    pl.kernel,
    out_type=jax.ShapeDtypeStruct((2, 16, ZROWS, 16), jnp.float32),
    mesh=_mesh,
    scratch_types=[
        pltpu.VMEM((ROWS_PER_TILE, 128), jnp.int32),
        pltpu.VMEM((128, 16), jnp.float32),
        pltpu.VMEM_SHARED((ACC_ROWS, 16), jnp.float32),
    ],
)
def _deg_kernel(dst_hbm, ones_hbm, z16_hbm, out_hbm, dstv, onesv, acc):
    cid = lax.axis_index("c")
    sid = lax.axis_index("s")
    pltpu.sync_copy(z16_hbm, acc.at[pl.ds(ZROWS * sid, ZROWS)])
    plsc.subcore_barrier()
    pltpu.sync_copy(ones_hbm, onesv)
    wid = cid * 16 + sid
    pltpu.sync_copy(dst_hbm.at[wid], dstv)

    def step(j, _):
        pltpu.sync_copy(onesv, acc.at[dstv.at[j]], add=True)
        return ()

    lax.fori_loop(0, ROWS_PER_TILE, step, ())
    plsc.subcore_barrier()
    pltpu.sync_copy(acc.at[pl.ds(ZROWS * sid, ZROWS)], out_hbm.at[cid, sid])


@functools.partial(
    pl.kernel,
    out_type=jax.ShapeDtypeStruct((2, 16, ZROWS, 128), jnp.float32),
    mesh=_mesh,
    scratch_types=[
        pltpu.VMEM((CH, 128), jnp.int32),
        pltpu.VMEM((CH, 128), jnp.int32),
        pltpu.VMEM((128, 128), jnp.float32),
        pltpu.VMEM((128, 128), jnp.float32),
        pltpu.VMEM_SHARED((ACC_ROWS, 128), jnp.float32),
        pltpu.SemaphoreType.DMA,
        pltpu.SemaphoreType.DMA,
    ],
)
def _scatter_kernel(g_hbm, src_hbm, dst_hbm, z128_hbm, out_hbm,
                    srcv, dstv, rows0, rows1, acc, sem0, sem1):
    cid = lax.axis_index("c")
    sid = lax.axis_index("s")
    pltpu.sync_copy(z128_hbm, acc.at[pl.ds(ZROWS * sid, ZROWS)])
    plsc.subcore_barrier()

    def outer(o, _):
        pltpu.sync_copy(src_hbm.at[cid, sid, o], srcv)
        pltpu.sync_copy(dst_hbm.at[sid, o], dstv)
        pltpu.async_copy(g_hbm.at[srcv.at[0]], rows0, sem0)

        def step(jj, _):
            j0 = 2 * jj
            pltpu.async_copy(g_hbm.at[srcv.at[j0 + 1]], rows1, sem1)
            pltpu.make_async_copy(g_hbm.at[srcv.at[j0]], rows0, sem0).wait()
            pltpu.sync_copy(rows0, acc.at[dstv.at[j0]], add=True)

            @pl.when(jj < CH // 2 - 1)
            def _():
                pltpu.async_copy(g_hbm.at[srcv.at[j0 + 2]], rows0, sem0)

            pltpu.make_async_copy(g_hbm.at[srcv.at[j0 + 1]], rows1, sem1).wait()
            pltpu.sync_copy(rows1, acc.at[dstv.at[j0 + 1]], add=True)
            return ()

        lax.fori_loop(0, CH // 2, step, ())
        return ()

    lax.fori_loop(0, OCH, outer, ())
    plsc.subcore_barrier()
    pltpu.sync_copy(acc.at[pl.ds(ZROWS * sid, ZROWS)], out_hbm.at[cid, sid])


@functools.partial(
    pl.kernel,
    out_type=jax.ShapeDtypeStruct((4, 64, 128), jnp.float32),
    mesh=_mesh,
    scratch_types=[
        pltpu.VMEM((80, 125), jnp.int32),
        pltpu.VMEM((125, 128), jnp.float32),
        pltpu.VMEM_SHARED((64, 128), jnp.float32),
    ],
)
def _pool_kernel(h_hbm, batch_hbm, z128_hbm, out_hbm, bv, rowsv, acc):
    cid = lax.axis_index("c")
    sid = lax.axis_index("s")
    pltpu.sync_copy(batch_hbm, bv)
    for cc in range(2):
        c = cid + 2 * cc
        @pl.when(sid == 0)
        def _():
            pltpu.sync_copy(z128_hbm.at[pl.ds(0, 64)], acc)

        plsc.subcore_barrier()
        for r in range(5):
            blk = 5 * sid + r
            pltpu.sync_copy(h_hbm.at[c, blk], rowsv)
            pltpu.sync_copy(rowsv, acc.at[bv.at[blk]], add=True)
        plsc.subcore_barrier()

        @pl.when(sid == 0)
        def _():
            pltpu.sync_copy(acc, out_hbm.at[c])

        plsc.subcore_barrier()


def _dinv_block(p_ref):
    deg = p_ref[0, :, 0:1] + p_ref[1, :, 0:1]
    return lax.rsqrt(deg)


def _tc1_body(p_ref, x_ref, w_ref, o_ref):
    dinv = _dinv_block(p_ref)
    o_ref[0] = dinv * jnp.dot(x_ref[...], w_ref[...],
                              preferred_element_type=jnp.float32)


def _mm_layer1(p, x, W1):
    return pl.pallas_call(
        _tc1_body,
        grid=(N // BN, 2),
        in_specs=[
            pl.BlockSpec((2, BN, 16), lambda i, j: (0, i, 0)),
            pl.BlockSpec((BN, 128), lambda i, j: (i, 0)),
            pl.BlockSpec((128, 128), lambda i, j: (0, j)),
        ],
        out_specs=pl.BlockSpec((1, BN, 128), lambda i, j: (j, i, 0)),
        out_shape=jax.ShapeDtypeStruct((2, N, 128), jnp.float32),
    )(p, x, W1)


def _tc_mid_body(nk, p_ref, raw_ref, b_ref, w_ref, o_ref):
    k = pl.program_id(2)
    dinv = _dinv_block(p_ref)
    h = jnp.maximum(dinv * raw_ref[0] + b_ref[0], 0.0)
    part = jnp.dot(h, w_ref[...], preferred_element_type=jnp.float32)

    @pl.when(k == 0)
    def _():
        o_ref[0] = part

    @pl.when(k > 0)
    def _():
        o_ref[0] += part

    @pl.when(k == nk - 1)
    def _():
        o_ref[0] = dinv * o_ref[0]


def _mm_mid(p, rawp, b_r, W, nj, nk):
    return pl.pallas_call(
        functools.partial(_tc_mid_body, nk),
        grid=(N // BN, nj, nk),
        in_specs=[
            pl.BlockSpec((2, BN, 16), lambda i, j, k: (0, i, 0)),
            pl.BlockSpec((1, BN, 128), lambda i, j, k: (k, i, 0)),
            pl.BlockSpec((1, 1, 128), lambda i, j, k: (k, 0, 0)),
            pl.BlockSpec((128, 128), lambda i, j, k: (k, j)),
        ],
        out_specs=pl.BlockSpec((1, BN, 128), lambda i, j, k: (j, i, 0)),
        out_shape=jax.ShapeDtypeStruct((nj, N, 128), jnp.float32),
    )(p, rawp, b_r, W)


def _tc_fin_body(p_ref, raw_ref, b_ref, o_ref):
    dinv = _dinv_block(p_ref)
    o_ref[0] = jnp.maximum(dinv * raw_ref[0] + b_ref[0], 0.0)


def _mm_finish(p, rawp, b_r):
    return pl.pallas_call(
        _tc_fin_body,
        grid=(N // BN, 4),
        in_specs=[
            pl.BlockSpec((2, BN, 16), lambda i, j: (0, i, 0)),
            pl.BlockSpec((1, BN, 128), lambda i, j: (j, i, 0)),
            pl.BlockSpec((1, 1, 128), lambda i, j: (j, 0, 0)),
        ],
        out_specs=pl.BlockSpec((1, BN, 128), lambda i, j: (j, i, 0)),
        out_shape=jax.ShapeDtypeStruct((4, N, 128), jnp.float32),
    )(p, rawp, b_r)


def _fc_body(pf_ref, wfc_ref, bfc_ref, wout_ref, bout_ref, o_ref):
    h = jnp.maximum(
        jnp.dot(pf_ref[...], wfc_ref[...], preferred_element_type=jnp.float32)
        + bfc_ref[...], 0.0)
    o_ref[...] = (jnp.dot(h, wout_ref[...], preferred_element_type=jnp.float32)
                  + bout_ref[...])


def _fc_head(pf, Wfc, bfc, Wout, bout):
    return pl.pallas_call(
        _fc_body,
        out_shape=jax.ShapeDtypeStruct((64, 256), jnp.float32),
    )(pf, Wfc, bfc.reshape(1, 512), Wout, bout.reshape(1, 256))


def kernel(x, edge_index, batch, W1, b1, W2, b2, W3, b3, Wfc, bfc, Wout, bout):
    loop = jnp.arange(N, dtype=jnp.int32)
    pad = EE_PAD - EE
    src0 = jnp.concatenate([edge_index[0], loop, jnp.zeros((pad,), jnp.int32)])
    dst0 = jnp.concatenate([edge_index[1], loop,
                            N + (jnp.arange(pad, dtype=jnp.int32) % 112)])
    src2 = jnp.stack([src0, src0 + N]).reshape(2, 16, OCH, CH, 128)
    dst2 = dst0.reshape(16, OCH, CH, 128)
    dst_deg = dst0[:EE_PAD0].reshape(32, ROWS_PER_TILE, 128)
    batch2d = batch.reshape(80, 125)
    ones16 = jnp.ones((128, 16), jnp.float32)
    z16 = jnp.zeros((ZROWS, 16), jnp.float32)
    z128 = jnp.zeros((ZROWS, 128), jnp.float32)

    p = _deg_kernel(dst_deg, ones16, z16).reshape(2, ACC_ROWS, 16)

    def conv_scatter(g):
        pairs = [_scatter_kernel(g[2 * q:2 * q + 2].reshape(2 * N, 128),
                                 src2, dst2, z128).reshape(2, ACC_ROWS, 128)
                 for q in range(g.shape[0] // 2)]
        return jnp.concatenate(pairs) if len(pairs) > 1 else pairs[0]

    g1 = _mm_layer1(p, x, W1)
    raw1 = conv_scatter(g1)
    g2 = _mm_mid(p, raw1, b1.reshape(2, 1, 128), W2, 2, 2)
    raw2 = conv_scatter(g2)
    g3 = _mm_mid(p, raw2, b2.reshape(2, 1, 128), W3, 4, 2)
    raw3 = conv_scatter(g3)
    h3 = _mm_finish(p, raw3, b3.reshape(4, 1, 128))

    pooled = _pool_kernel(h3.reshape(4, 80, 125, 128), batch2d, z128)
    embedding = jnp.transpose(pooled, (1, 0, 2)).reshape(64, 512)
    output = _fc_head(embedding, Wfc, bfc, Wout, bout)
    return (embedding, output)


# speedup vs baseline: 5.2117x; 1.0388x over previous
import functools

import jax
import jax.numpy as jnp
from jax import lax
from jax.experimental import pallas as pl
from jax.experimental.pallas import tpu as pltpu
from jax.experimental.pallas import tpu_sc as plsc

N = 10000
E = 320000
EE = E + N
ROWS = 2592
EE_PAD0 = ROWS * 128
ROWS_PER_TILE = ROWS // 32
SLAB = 96
EE_PAD = 3456 * SLAB
OCH = 9
CH = 24
ACC_ROWS = 10112
ZROWS = ACC_ROWS // 16
BN = 1000

_mesh = plsc.VectorSubcoreMesh(core_axis_name="c", subcore_axis_name="s")


@functools.partial(
    pl.kernel,
    out_type=jax.ShapeDtypeStruct((2, 16, ZROWS, 16), jnp.float32),
    mesh=_mesh,
    scratch_types=[
        pltpu.VMEM((ROWS_PER_TILE, 128), jnp.int32),
        pltpu.VMEM((128, 16), jnp.float32),
        pltpu.VMEM_SHARED((ACC_ROWS, 16), jnp.float32),
    ],
)
def _deg_kernel(dst_hbm, ones_hbm, z16_hbm, out_hbm, dstv, onesv, acc):
    cid = lax.axis_index("c")
    sid = lax.axis_index("s")
    pltpu.sync_copy(z16_hbm, acc.at[pl.ds(ZROWS * sid, ZROWS)])
    plsc.subcore_barrier()
    pltpu.sync_copy(ones_hbm, onesv)
    wid = cid * 16 + sid
    pltpu.sync_copy(dst_hbm.at[wid], dstv)

    def step(j, _):
        pltpu.sync_copy(onesv, acc.at[dstv.at[j]], add=True)
        return ()

    lax.fori_loop(0, ROWS_PER_TILE, step, ())
    plsc.subcore_barrier()
    pltpu.sync_copy(acc.at[pl.ds(ZROWS * sid, ZROWS)], out_hbm.at[cid, sid])


@functools.partial(
    pl.kernel,
    out_type=jax.ShapeDtypeStruct((2, 16, ZROWS, 128), jnp.float32),
    mesh=_mesh,
    scratch_types=[
        pltpu.VMEM((CH, SLAB), jnp.int32),
        pltpu.VMEM((CH, SLAB), jnp.int32),
        pltpu.VMEM((SLAB, 128), jnp.float32),
        pltpu.VMEM((SLAB, 128), jnp.float32),
        pltpu.VMEM((SLAB, 128), jnp.float32),
        pltpu.SemaphoreType.DMA,
        pltpu.SemaphoreType.DMA,
        pltpu.SemaphoreType.DMA,
        pltpu.SemaphoreType.DMA,
        pltpu.SemaphoreType.DMA,
        pltpu.SemaphoreType.DMA,
        pltpu.VMEM_SHARED((ACC_ROWS, 128), jnp.float32),
    ],
)
def _scatter_kernel(g_hbm, src_hbm, dst_hbm, z128_hbm, out_hbm,
                    srcv, dstv, r0, r1, r2, sg0, sg1, sg2, ss0, ss1, ss2, acc):
    cid = lax.axis_index("c")
    sid = lax.axis_index("s")
    rows = (r0, r1, r2)
    sgs = (sg0, sg1, sg2)
    sss = (ss0, ss1, ss2)
    pltpu.sync_copy(z128_hbm, acc.at[pl.ds(ZROWS * sid, ZROWS)])
    plsc.subcore_barrier()

    def outer(o, _):
        pltpu.sync_copy(src_hbm.at[cid, sid, o], srcv)
        pltpu.sync_copy(dst_hbm.at[sid, o], dstv)
        pltpu.async_copy(g_hbm.at[srcv.at[0]], rows[0], sgs[0])
        pltpu.async_copy(g_hbm.at[srcv.at[1]], rows[1], sgs[1])

        def group(t, _):
            for u in range(3):
                j = 3 * t + u
                bp = (u + 2) % 3

                @pl.when(j >= 1)
                def _():
                    pltpu.make_async_copy(
                        rows[bp], acc.at[dstv.at[j - 1]], sss[bp]).wait()

                @pl.when(j + 2 < CH)
                def _():
                    pltpu.async_copy(g_hbm.at[srcv.at[j + 2]], rows[bp], sgs[bp])

                pltpu.make_async_copy(g_hbm.at[srcv.at[j]], rows[u], sgs[u]).wait()
                pltpu.async_copy(rows[u], acc.at[dstv.at[j]], sss[u], add=True)
            return ()

        lax.fori_loop(0, CH // 3, group, ())
        pltpu.make_async_copy(rows[2], acc.at[dstv.at[CH - 1]], sss[2]).wait()
        return ()

    lax.fori_loop(0, OCH, outer, ())
    plsc.subcore_barrier()
    pltpu.sync_copy(acc.at[pl.ds(ZROWS * sid, ZROWS)], out_hbm.at[cid, sid])


@functools.partial(
    pl.kernel,
    out_type=jax.ShapeDtypeStruct((4, 64, 128), jnp.float32),
    mesh=_mesh,
    scratch_types=[
        pltpu.VMEM((80, 125), jnp.int32),
        pltpu.VMEM((125, 128), jnp.float32),
        pltpu.VMEM_SHARED((64, 128), jnp.float32),
    ],
)
def _pool_kernel(h_hbm, batch_hbm, z128_hbm, out_hbm, bv, rowsv, acc):
    cid = lax.axis_index("c")
    sid = lax.axis_index("s")
    pltpu.sync_copy(batch_hbm, bv)
    for cc in range(2):
        c = cid + 2 * cc
        @pl.when(sid == 0)
        def _():
            pltpu.sync_copy(z128_hbm.at[pl.ds(0, 64)], acc)

        plsc.subcore_barrier()
        for r in range(5):
            blk = 5 * sid + r
            pltpu.sync_copy(h_hbm.at[c, blk], rowsv)
            pltpu.sync_copy(rowsv, acc.at[bv.at[blk]], add=True)
        plsc.subcore_barrier()

        @pl.when(sid == 0)
        def _():
            pltpu.sync_copy(acc, out_hbm.at[c])

        plsc.subcore_barrier()


def _dinv_block(p_ref):
    deg = p_ref[0, :, 0:1] + p_ref[1, :, 0:1]
    return lax.rsqrt(deg)


def _tc1_body(p_ref, x_ref, w_ref, o_ref):
    dinv = _dinv_block(p_ref)
    o_ref[0] = dinv * jnp.dot(x_ref[...], w_ref[...],
                              preferred_element_type=jnp.float32)


def _mm_layer1(p, x, W1):
    return pl.pallas_call(
        _tc1_body,
        grid=(N // BN, 2),
        in_specs=[
            pl.BlockSpec((2, BN, 16), lambda i, j: (0, i, 0)),
            pl.BlockSpec((BN, 128), lambda i, j: (i, 0)),
            pl.BlockSpec((128, 128), lambda i, j: (0, j)),
        ],
        out_specs=pl.BlockSpec((1, BN, 128), lambda i, j: (j, i, 0)),
        out_shape=jax.ShapeDtypeStruct((2, N, 128), jnp.float32),
    )(p, x, W1)


def _tc_mid_body(nk, p_ref, raw_ref, b_ref, w_ref, o_ref):
    k = pl.program_id(2)
    dinv = _dinv_block(p_ref)
    h = jnp.maximum(dinv * raw_ref[0] + b_ref[0], 0.0)
    part = jnp.dot(h, w_ref[...], preferred_element_type=jnp.float32)

    @pl.when(k == 0)
    def _():
        o_ref[0] = part

    @pl.when(k > 0)
    def _():
        o_ref[0] += part

    @pl.when(k == nk - 1)
    def _():
        o_ref[0] = dinv * o_ref[0]


def _mm_mid(p, rawp, b_r, W, nj, nk):
    return pl.pallas_call(
        functools.partial(_tc_mid_body, nk),
        grid=(N // BN, nj, nk),
        in_specs=[
            pl.BlockSpec((2, BN, 16), lambda i, j, k: (0, i, 0)),
            pl.BlockSpec((1, BN, 128), lambda i, j, k: (k, i, 0)),
            pl.BlockSpec((1, 1, 128), lambda i, j, k: (k, 0, 0)),
            pl.BlockSpec((128, 128), lambda i, j, k: (k, j)),
        ],
        out_specs=pl.BlockSpec((1, BN, 128), lambda i, j, k: (j, i, 0)),
        out_shape=jax.ShapeDtypeStruct((nj, N, 128), jnp.float32),
    )(p, rawp, b_r, W)


def _tc_fin_body(p_ref, raw_ref, b_ref, o_ref):
    dinv = _dinv_block(p_ref)
    o_ref[0] = jnp.maximum(dinv * raw_ref[0] + b_ref[0], 0.0)


def _mm_finish(p, rawp, b_r):
    return pl.pallas_call(
        _tc_fin_body,
        grid=(N // BN, 4),
        in_specs=[
            pl.BlockSpec((2, BN, 16), lambda i, j: (0, i, 0)),
            pl.BlockSpec((1, BN, 128), lambda i, j: (j, i, 0)),
            pl.BlockSpec((1, 1, 128), lambda i, j: (j, 0, 0)),
        ],
        out_specs=pl.BlockSpec((1, BN, 128), lambda i, j: (j, i, 0)),
        out_shape=jax.ShapeDtypeStruct((4, N, 128), jnp.float32),
    )(p, rawp, b_r)


def _fc_body(pf_ref, wfc_ref, bfc_ref, wout_ref, bout_ref, o_ref):
    h = jnp.maximum(
        jnp.dot(pf_ref[...], wfc_ref[...], preferred_element_type=jnp.float32)
        + bfc_ref[...], 0.0)
    o_ref[...] = (jnp.dot(h, wout_ref[...], preferred_element_type=jnp.float32)
                  + bout_ref[...])


def _fc_head(pf, Wfc, bfc, Wout, bout):
    return pl.pallas_call(
        _fc_body,
        out_shape=jax.ShapeDtypeStruct((64, 256), jnp.float32),
    )(pf, Wfc, bfc.reshape(1, 512), Wout, bout.reshape(1, 256))


def kernel(x, edge_index, batch, W1, b1, W2, b2, W3, b3, Wfc, bfc, Wout, bout):
    loop = jnp.arange(N, dtype=jnp.int32)
    pad = EE_PAD - EE
    src0 = jnp.concatenate([edge_index[0], loop, jnp.zeros((pad,), jnp.int32)])
    dst0 = jnp.concatenate([edge_index[1], loop,
                            N + (jnp.arange(pad, dtype=jnp.int32) % 112)])
    src2 = jnp.stack([src0, src0 + N]).reshape(2, 16, OCH, CH, SLAB)
    dst2 = dst0.reshape(16, OCH, CH, SLAB)
    dst_deg = dst0[:EE_PAD0].reshape(32, ROWS_PER_TILE, 128)
    batch2d = batch.reshape(80, 125)
    ones16 = jnp.ones((128, 16), jnp.float32)
    z16 = jnp.zeros((ZROWS, 16), jnp.float32)
    z128 = jnp.zeros((ZROWS, 128), jnp.float32)

    p = _deg_kernel(dst_deg, ones16, z16).reshape(2, ACC_ROWS, 16)

    def conv_scatter(g):
        pairs = [_scatter_kernel(g[2 * q:2 * q + 2].reshape(2 * N, 128),
                                 src2, dst2, z128).reshape(2, ACC_ROWS, 128)
                 for q in range(g.shape[0] // 2)]
        return jnp.concatenate(pairs) if len(pairs) > 1 else pairs[0]

    g1 = _mm_layer1(p, x, W1)
    raw1 = conv_scatter(g1)
    g2 = _mm_mid(p, raw1, b1.reshape(2, 1, 128), W2, 2, 2)
    raw2 = conv_scatter(g2)
    g3 = _mm_mid(p, raw2, b2.reshape(2, 1, 128), W3, 4, 2)
    raw3 = conv_scatter(g3)
    h3 = _mm_finish(p, raw3, b3.reshape(4, 1, 128))

    pooled = _pool_kernel(h3.reshape(4, 80, 125, 128), batch2d, z128)
    embedding = jnp.transpose(pooled, (1, 0, 2)).reshape(64, 512)
    output = _fc_head(embedding, Wfc, bfc, Wout, bout)
    return (embedding, output)
